# baseline (speedup 1.0000x reference)
import functools

import jax
import jax.numpy as jnp
from jax import lax
from jax.experimental import pallas as pl
from jax.experimental.pallas import tpu as pltpu
from jax.experimental.pallas import tpu_sc as plsc

N = 2048
D = 128
NHEAD = 8
NHID = 128
NCLASS = 10
ALPHA = 0.2

RB = 256
CB = 256
NRB = N // RB
NCB = N // CB

NC = 2
NS = 16
L = 16
NW = NC * NS
E = 32768
E2 = E + N
EPT = E2 // NW
NGT = EPT // L
GCH = 64
NCH = EPT // GCH
RPT = N // NS

NH_TC = 6
NH_SC = NHEAD - NH_TC


def _leaky(v):
    return jnp.maximum(v, ALPHA * v)


def _sc_claim_body(src_hbm, dst_hbm, claim_hbm, sv, dv, kv_all, sem):
    c = lax.axis_index("c")
    tid = lax.axis_index("s")
    wid = tid * NC + c
    base = wid * EPT
    pltpu.sync_copy(src_hbm.at[pl.ds(base, EPT)], sv)
    pltpu.sync_copy(dst_hbm.at[pl.ds(base, EPT)], dv)

    def prep_g(g, carry):
        sl = pl.ds(g * L, L)
        kv_all[sl] = base + g * L + lax.iota(jnp.int32, L)
        return carry

    lax.fori_loop(0, NGT, prep_g, 0)

    def issue_g(g, carry):
        sl = pl.ds(g * L, L)
        cellv = sv[sl] * N + dv[sl]
        pltpu.async_copy(kv_all.at[sl], claim_hbm.at[cellv], sem)
        return carry

    lax.fori_loop(0, NGT, issue_g, 0)

    def drain_g(g, carry):
        sl = pl.ds(g * L, L)
        cellv = sv[sl] * N + dv[sl]
        pltpu.make_async_copy(kv_all.at[sl], claim_hbm.at[cellv], sem).wait()
        return carry

    lax.fori_loop(0, NGT, drain_g, 0)


def _sc_agg_body(src_hbm, dst_hbm, claim_hbm, inadj_hbm, hf_hbm, asf_hbm,
                 adf_hbm, zn_hbm, numf_hbm, denf_hbm,
                 sv, dv, cbuf, got, adjv, geff, shead, dhead, rows0, asb0,
                 adb0, wbuf0, rows1, asb1, adb1, wbuf1, num_sh, den_sh, sem0,
                 sem1, sema0, sema1):
    c = lax.axis_index("c")
    tid = lax.axis_index("s")
    wid = tid * NC + c
    base = wid * EPT
    pltpu.sync_copy(src_hbm.at[pl.ds(base, EPT)], sv)
    pltpu.sync_copy(dst_hbm.at[pl.ds(base, EPT)], dv)

    def cells_g(g, carry):
        sl = pl.ds(g * L, L)
        cbuf[sl] = sv[sl] * N + dv[sl]
        return carry

    lax.fori_loop(0, NGT, cells_g, 0)

    for t in range(EPT // GCH):
        wsl = pl.ds(t * GCH, GCH)
        pltpu.async_copy(claim_hbm.at[cbuf.at[wsl]], got.at[wsl], sem0).wait()
        pltpu.async_copy(inadj_hbm.at[cbuf.at[wsl]], adjv.at[wsl], sem0).wait()

    def geff_g(g, carry):
        sl = pl.ds(g * L, L)
        kv = base + g * L + lax.iota(jnp.int32, L)
        canon = got[sl] == kv
        svv = sv[sl]
        geff[sl] = jnp.where(canon,
                             adjv[sl] + jnp.where(svv == dv[sl], 1.0, 0.0),
                             0.0)
        return carry

    lax.fori_loop(0, NGT, geff_g, 0)

    myrows = pl.ds(tid * RPT, RPT)

    def head_body(i, carry):
        def dh_g(g, carry2):
            sl = pl.ds(g * L, L)
            shead[sl] = sv[sl] + (i + NH_TC) * N
            dhead[sl] = dv[sl] + (i + NH_TC) * N
            return carry2

        lax.fori_loop(0, NGT, dh_g, 0)

        pltpu.sync_copy(zn_hbm.at[myrows], num_sh.at[myrows])
        pltpu.sync_copy(zn_hbm.at[0], den_sh.at[pl.ds(tid * RPT, RPT)])
        plsc.subcore_barrier()

        def _issue(off, rows, asb, adb, semx):
            wsl = pl.ds(off, GCH)
            pltpu.async_copy(hf_hbm.at[dhead.at[wsl]], rows, semx)
            pltpu.async_copy(adf_hbm.at[dhead.at[wsl]], adb, semx)
            pltpu.async_copy(asf_hbm.at[shead.at[wsl]], asb, semx)

        def _process(off, rows, asb, adb, wbuf, semx, semax):
            wsl = pl.ds(off, GCH)
            pltpu.make_async_copy(hf_hbm.at[dhead.at[wsl]], rows, semx).wait()
            pltpu.make_async_copy(adf_hbm.at[dhead.at[wsl]], adb, semx).wait()
            pltpu.make_async_copy(asf_hbm.at[shead.at[wsl]], asb, semx).wait()
            adds = []
            for g in range(GCH // L):
                sl = pl.ds(off + g * L, L)
                gsl = pl.ds(g * L, L)
                w = jnp.exp(geff[sl] * _leaky(asb[gsl] + adb[gsl])) - 1.0
                wbuf[gsl] = w
                for eo in range(L):
                    ei = g * L + eo
                    wb = jnp.broadcast_to(w[eo], (L,))
                    for j in range(D // L):
                        rows[ei, pl.ds(j * L, L)] = (
                            rows[ei, pl.ds(j * L, L)] * wb)
                svv = sv[sl]
                adds.append(pltpu.async_copy(rows.at[gsl], num_sh.at[svv],
                                             semax, add=True))
                adds.append(pltpu.async_copy(wbuf.at[gsl], den_sh.at[svv],
                                             semax, add=True))
            for a in adds:
                a.wait()

        def chunk_body(q, carry2):
            @pl.when(jnp.logical_and(q < NCH, q % 2 == 0))
            def _i0():
                _issue(q * GCH, rows0, asb0, adb0, sem0)

            @pl.when(jnp.logical_and(q < NCH, q % 2 == 1))
            def _i1():
                _issue(q * GCH, rows1, asb1, adb1, sem1)

            @pl.when(jnp.logical_and(q > 0, (q - 1) % 2 == 0))
            def _p0():
                _process((q - 1) * GCH, rows0, asb0, adb0, wbuf0, sem0, sema0)

            @pl.when(jnp.logical_and(q > 0, (q - 1) % 2 == 1))
            def _p1():
                _process((q - 1) * GCH, rows1, asb1, adb1, wbuf1, sem1, sema1)

            return carry2

        lax.fori_loop(0, NCH + 1, chunk_body, 0)
        plsc.subcore_barrier()
        obase = (c * NH_SC + i) * N + tid * RPT
        pltpu.sync_copy(num_sh.at[myrows], numf_hbm.at[pl.ds(obase, RPT)])
        pltpu.sync_copy(den_sh.at[pl.ds(tid * RPT, RPT)],
                        denf_hbm.at[pl.ds(obase, RPT)])
        plsc.subcore_barrier()
        return carry

    lax.fori_loop(0, NH_SC, head_body, 0)


def _sc_aggregate(srcv, dstv, in_adj_flat, h, as_, ad_):
    mesh = plsc.VectorSubcoreMesh(core_axis_name="c", subcore_axis_name="s")
    claim_fn = pl.kernel(
        _sc_claim_body,
        out_type=jax.ShapeDtypeStruct((N * N,), jnp.int32),
        mesh=mesh,
        scratch_types=[
            pltpu.VMEM((EPT,), jnp.int32),
            pltpu.VMEM((EPT,), jnp.int32),
            pltpu.VMEM((EPT,), jnp.int32),
            pltpu.SemaphoreType.DMA,
        ],
    )
    claim = claim_fn(srcv, dstv)

    hf = h.reshape(NHEAD * N, D)
    asf = as_.reshape(NHEAD * N)
    adf = ad_.reshape(NHEAD * N)
    zn = jnp.zeros((N, D), jnp.float32)

    agg_fn = pl.kernel(
        _sc_agg_body,
        out_type=[
            jax.ShapeDtypeStruct((NC * NH_SC * N, D), jnp.float32),
            jax.ShapeDtypeStruct((NC * NH_SC * N,), jnp.float32),
        ],
        mesh=mesh,
        scratch_types=[
            pltpu.VMEM((EPT,), jnp.int32),
            pltpu.VMEM((EPT,), jnp.int32),
            pltpu.VMEM((EPT,), jnp.int32),
            pltpu.VMEM((EPT,), jnp.int32),
            pltpu.VMEM((EPT,), jnp.float32),
            pltpu.VMEM((EPT,), jnp.float32),
            pltpu.VMEM((EPT,), jnp.int32),
            pltpu.VMEM((EPT,), jnp.int32),
            pltpu.VMEM((GCH, D), jnp.float32),
            pltpu.VMEM((GCH,), jnp.float32),
            pltpu.VMEM((GCH,), jnp.float32),
            pltpu.VMEM((GCH,), jnp.float32),
            pltpu.VMEM((GCH, D), jnp.float32),
            pltpu.VMEM((GCH,), jnp.float32),
            pltpu.VMEM((GCH,), jnp.float32),
            pltpu.VMEM((GCH,), jnp.float32),
            pltpu.VMEM_SHARED((N, D), jnp.float32),
            pltpu.VMEM_SHARED((N,), jnp.float32),
            pltpu.SemaphoreType.DMA,
            pltpu.SemaphoreType.DMA,
            pltpu.SemaphoreType.DMA,
            pltpu.SemaphoreType.DMA,
        ],
    )
    numf, denf = agg_fn(srcv, dstv, claim, in_adj_flat, hf, asf, adf, zn)
    return (numf.reshape(NC, NH_SC, N, D), denf.reshape(NC, NH_SC, N))


def _proj_kernel(x_ref, w_ref, a1_ref, a2_ref, h_ref, as_ref, ad_ref, hsum_ref):
    r = pl.program_id(0)
    xb = x_ref[...]
    as_rows = []
    ad_rows = []
    hs = []
    for i in range(NHEAD):
        h = jnp.dot(xb, w_ref[i], preferred_element_type=jnp.float32)
        h_ref[i] = h
        as_rows.append(lax.dot_general(
            a1_ref[i][None, :], h, (((1,), (1,)), ((), ())),
            preferred_element_type=jnp.float32))
        ad_rows.append(lax.dot_general(
            a2_ref[i][None, :], h, (((1,), (1,)), ((), ())),
            preferred_element_type=jnp.float32))
        hs.append(jnp.sum(h, axis=0, keepdims=True))
    as_ref[...] = jnp.concatenate(as_rows, axis=0)
    ad_ref[...] = jnp.concatenate(ad_rows, axis=0)
    part = jnp.concatenate(hs, axis=0)

    @pl.when(r == 0)
    def _init():
        hsum_ref[...] = part

    @pl.when(r != 0)
    def _acc():
        hsum_ref[...] += part


def _heads_kernel(in_adj_ref, h_ref, as_ref, ad_ref, hsum_ref, b_ref,
                  h1a_ref, acc_ref, den_ref):
    r = pl.program_id(0)
    c = pl.program_id(1)
    rows = r * RB + jax.lax.broadcasted_iota(jnp.int32, (RB, CB), 0)
    cols = c * CB + jax.lax.broadcasted_iota(jnp.int32, (RB, CB), 1)
    adj = in_adj_ref[...] + jnp.where(rows == cols, 1.0, 0.0)

    dens = []
    for i in range(NH_TC):
        e = _leaky(as_ref[i][:, None] + ad_ref[i][None, :])
        w = jnp.exp(adj * e) - 1.0
        dens.append(jnp.sum(w, axis=1, keepdims=True))
        contrib = jnp.dot(w, h_ref[i], preferred_element_type=jnp.float32)

        @pl.when(c == 0)
        def _init(i=i, contrib=contrib):
            acc_ref[i] = contrib

        @pl.when(c != 0)
        def _acc(i=i, contrib=contrib):
            acc_ref[i] += contrib

    den_part = jnp.concatenate(dens, axis=1)

    @pl.when(c == 0)
    def _dinit():
        den_ref[...] = den_part

    @pl.when(c != 0)
    def _dacc():
        den_ref[...] += den_part

    @pl.when(c == NCB - 1)
    def _finalize():
        outs = []
        for i in range(NH_TC):
            numer = hsum_ref[i][None, :] + acc_ref[i]
            den = float(N) + den_ref[:, i][:, None]
            o = numer / den + b_ref[i][None, :]
            outs.append(jnp.where(o > 0, o, jnp.exp(o) - 1.0))
        h1a_ref[...] = jnp.concatenate(outs, axis=1)


def _hfin_kernel(num_ref, den_ref, hsum_ref, b_ref, h1_ref):
    outs = []
    for i in range(NH_SC):
        numer = hsum_ref[NH_TC + i][None, :] + num_ref[0, i] + num_ref[1, i]
        den = float(N) + den_ref[0, i, :] + den_ref[1, i, :]
        o = numer / den[:, None] + b_ref[NH_TC + i][None, :]
        outs.append(jnp.where(o > 0, o, jnp.exp(o) - 1.0))
    h1_ref[...] = jnp.concatenate(outs, axis=1)


def _out_proj_kernel(h1a_ref, h1b_ref, wa_ref, wb_ref, h2_ref, hsum2_ref):
    h2 = (jnp.dot(h1a_ref[...], wa_ref[...], preferred_element_type=jnp.float32)
          + jnp.dot(h1b_ref[...], wb_ref[...],
                    preferred_element_type=jnp.float32))
    h2_ref[...] = h2
    hsum2_ref[0] = jnp.sum(h2, axis=0, keepdims=True)


def _final_kernel(in_adj_ref, h2r_ref, h2c_ref, hsum2_ref, a1_ref, a2_ref,
                  b_ref, adj_ref, out_ref, acc_ref, den_ref):
    r = pl.program_id(0)
    c = pl.program_id(1)
    rows = r * RB + jax.lax.broadcasted_iota(jnp.int32, (RB, CB), 0)
    cols = c * CB + jax.lax.broadcasted_iota(jnp.int32, (RB, CB), 1)
    adj = in_adj_ref[...] + jnp.where(rows == cols, 1.0, 0.0)
    adj_ref[...] = adj

    h2r = h2r_ref[...]
    h2c = h2c_ref[...]
    asr = jnp.dot(h2r, a1_ref[...], preferred_element_type=jnp.float32)
    adc = jnp.dot(h2c, a2_ref[...], preferred_element_type=jnp.float32)
    e = _leaky(asr + adc[:, 0][None, :])
    w = jnp.exp(adj * e) - 1.0
    den_part = jnp.sum(w, axis=1, keepdims=True)
    contrib = jnp.dot(w, h2c, preferred_element_type=jnp.float32)

    @pl.when(c == 0)
    def _init():
        acc_ref[...] = contrib
        den_ref[...] = jnp.broadcast_to(den_part, (RB, 128))

    @pl.when(c != 0)
    def _acc():
        acc_ref[...] += contrib
        den_ref[...] += jnp.broadcast_to(den_part, (RB, 128))

    @pl.when(c == NCB - 1)
    def _finalize():
        hsum2 = jnp.sum(hsum2_ref[...], axis=0)
        numer = hsum2 + acc_ref[...]
        den = float(N) + den_ref[:, 0][:, None]
        o = numer / den + b_ref[...]
        lane = jax.lax.broadcasted_iota(jnp.int32, (RB, 128), 1)
        valid = lane < NCLASS
        om = jnp.where(valid, o, -jnp.inf)
        mx = jnp.max(om, axis=1, keepdims=True)
        ex = jnp.where(valid, jnp.exp(om - mx), 0.0)
        lse = jnp.log(jnp.sum(ex, axis=1, keepdims=True)) + mx
        out_ref[...] = jnp.where(valid, o - lse, 0.0)


def kernel(x, in_adj, edge_index, W_heads, a_heads, b_heads, W_out, a_out, b_out):
    a1 = a_heads[:, :D, 0]
    a2 = a_heads[:, D:, 0]

    h, as_, ad_, hsum = pl.pallas_call(
        _proj_kernel,
        grid=(NRB,),
        in_specs=[
            pl.BlockSpec((RB, D), lambda r: (r, 0)),
            pl.BlockSpec((NHEAD, D, D), lambda r: (0, 0, 0)),
            pl.BlockSpec((NHEAD, D), lambda r: (0, 0)),
            pl.BlockSpec((NHEAD, D), lambda r: (0, 0)),
        ],
        out_specs=[
            pl.BlockSpec((NHEAD, RB, D), lambda r: (0, r, 0)),
            pl.BlockSpec((NHEAD, RB), lambda r: (0, r)),
            pl.BlockSpec((NHEAD, RB), lambda r: (0, r)),
            pl.BlockSpec((NHEAD, D), lambda r: (0, 0)),
        ],
        out_shape=[
            jax.ShapeDtypeStruct((NHEAD, N, D), jnp.float32),
            jax.ShapeDtypeStruct((NHEAD, N), jnp.float32),
            jax.ShapeDtypeStruct((NHEAD, N), jnp.float32),
            jax.ShapeDtypeStruct((NHEAD, D), jnp.float32),
        ],
        compiler_params=pltpu.CompilerParams(
            dimension_semantics=("arbitrary",)),
    )(x, W_heads, a1, a2)

    loop = jnp.arange(N, dtype=jnp.int32)
    srcv = jnp.concatenate([edge_index[0].astype(jnp.int32), loop])
    dstv = jnp.concatenate([edge_index[1].astype(jnp.int32), loop])
    num, den = _sc_aggregate(srcv, dstv, in_adj.reshape(-1), h, as_, ad_)

    h1a = pl.pallas_call(
        _heads_kernel,
        grid=(NRB, NCB),
        in_specs=[
            pl.BlockSpec((RB, CB), lambda r, c: (r, c)),
            pl.BlockSpec((NH_TC, CB, D), lambda r, c: (0, c, 0)),
            pl.BlockSpec((NHEAD, RB), lambda r, c: (0, r)),
            pl.BlockSpec((NHEAD, CB), lambda r, c: (0, c)),
            pl.BlockSpec((NHEAD, D), lambda r, c: (0, 0)),
            pl.BlockSpec((NHEAD, D), lambda r, c: (0, 0)),
        ],
        out_specs=pl.BlockSpec((RB, NH_TC * D), lambda r, c: (r, 0)),
        out_shape=jax.ShapeDtypeStruct((N, NH_TC * D), jnp.float32),
        scratch_shapes=[
            pltpu.VMEM((NH_TC, RB, D), jnp.float32),
            pltpu.VMEM((RB, NH_TC), jnp.float32),
        ],
        compiler_params=pltpu.CompilerParams(
            dimension_semantics=("parallel", "arbitrary")),
    )(in_adj, h, as_, ad_, hsum, b_heads)

    h1b = pl.pallas_call(
        _hfin_kernel,
        grid=(NRB,),
        in_specs=[
            pl.BlockSpec((NC, NH_SC, RB, D), lambda r: (0, 0, r, 0)),
            pl.BlockSpec((NC, NH_SC, RB), lambda r: (0, 0, r)),
            pl.BlockSpec((NHEAD, D), lambda r: (0, 0)),
            pl.BlockSpec((NHEAD, D), lambda r: (0, 0)),
        ],
        out_specs=pl.BlockSpec((RB, NH_SC * D), lambda r: (r, 0)),
        out_shape=jax.ShapeDtypeStruct((N, NH_SC * D), jnp.float32),
        compiler_params=pltpu.CompilerParams(
            dimension_semantics=("parallel",)),
    )(num, den, hsum, b_heads)

    wout_pad = jnp.zeros((NHEAD * D, 128), jnp.float32).at[:, :NCLASS].set(W_out)
    wa = wout_pad[:NH_TC * D]
    wb = wout_pad[NH_TC * D:]
    a1o = jnp.zeros((128, 1), jnp.float32).at[:NCLASS, 0].set(a_out[:NCLASS, 0])
    a2o = jnp.zeros((128, 1), jnp.float32).at[:NCLASS, 0].set(a_out[NCLASS:, 0])
    bo = jnp.zeros((1, 128), jnp.float32).at[0, :NCLASS].set(b_out)

    h2, hsum2 = pl.pallas_call(
        _out_proj_kernel,
        grid=(NRB,),
        in_specs=[
            pl.BlockSpec((RB, NH_TC * D), lambda r: (r, 0)),
            pl.BlockSpec((RB, NH_SC * D), lambda r: (r, 0)),
            pl.BlockSpec((NH_TC * D, 128), lambda r: (0, 0)),
            pl.BlockSpec((NH_SC * D, 128), lambda r: (0, 0)),
        ],
        out_specs=[
            pl.BlockSpec((RB, 128), lambda r: (r, 0)),
            pl.BlockSpec((1, 1, 128), lambda r: (r, 0, 0)),
        ],
        out_shape=[
            jax.ShapeDtypeStruct((N, 128), jnp.float32),
            jax.ShapeDtypeStruct((NRB, 1, 128), jnp.float32),
        ],
        compiler_params=pltpu.CompilerParams(
            dimension_semantics=("arbitrary",)),
    )(h1a, h1b, wa, wb)

    adj, out_pad = pl.pallas_call(
        _final_kernel,
        grid=(NRB, NCB),
        in_specs=[
            pl.BlockSpec((RB, CB), lambda r, c: (r, c)),
            pl.BlockSpec((RB, 128), lambda r, c: (r, 0)),
            pl.BlockSpec((CB, 128), lambda r, c: (c, 0)),
            pl.BlockSpec((NRB, 1, 128), lambda r, c: (0, 0, 0)),
            pl.BlockSpec((128, 1), lambda r, c: (0, 0)),
            pl.BlockSpec((128, 1), lambda r, c: (0, 0)),
            pl.BlockSpec((1, 128), lambda r, c: (0, 0)),
        ],
        out_specs=[
            pl.BlockSpec((RB, CB), lambda r, c: (r, c)),
            pl.BlockSpec((RB, 128), lambda r, c: (r, 0)),
        ],
        out_shape=[
            jax.ShapeDtypeStruct((N, N), jnp.float32),
            jax.ShapeDtypeStruct((N, 128), jnp.float32),
        ],
        scratch_shapes=[
            pltpu.VMEM((RB, 128), jnp.float32),
            pltpu.VMEM((RB, 128), jnp.float32),
        ],
        compiler_params=pltpu.CompilerParams(
            dimension_semantics=("parallel", "arbitrary")),
    )(in_adj, h2, h2, hsum2, a1o, a2o, bo)

    return out_pad[:, :NCLASS], adj, x


# speedup vs baseline: 1.0005x; 1.0005x over previous
import jax
import jax.numpy as jnp
from jax import lax
from jax.experimental import pallas as pl
from jax.experimental.pallas import tpu as pltpu
from jax.experimental.pallas import tpu_sc as plsc

N = 2048
D = 128
NHEAD = 8
NHID = 128
NCLASS = 10
ALPHA = 0.2

RB = 256
CB = 256
NRB = N // RB
NCB = N // CB

NC = 2
NS = 16
L = 16
NW = NC * NS
E = 32768
E2 = E + N
EPT = E2 // NW
NGT = EPT // L
GCH = 64
NCH = EPT // GCH
RPT = N // NS

NH_TC = 6
NH_SC = NHEAD - NH_TC


def _leaky(v):
    return jnp.maximum(v, ALPHA * v)


def _sc_claim_body(src_hbm, dst_hbm, claim_hbm, sv, dv, kv_all, sem):
    c = lax.axis_index("c")
    tid = lax.axis_index("s")
    wid = tid * NC + c
    base = wid * EPT
    pltpu.sync_copy(src_hbm.at[pl.ds(base, EPT)], sv)
    pltpu.sync_copy(dst_hbm.at[pl.ds(base, EPT)], dv)

    def prep_g(g, carry):
        sl = pl.ds(g * L, L)
        kv_all[sl] = base + g * L + lax.iota(jnp.int32, L)
        return carry

    lax.fori_loop(0, NGT, prep_g, 0)

    def issue_g(g, carry):
        sl = pl.ds(g * L, L)
        cellv = sv[sl] * N + dv[sl]
        pltpu.async_copy(kv_all.at[sl], claim_hbm.at[cellv], sem)
        return carry

    lax.fori_loop(0, NGT, issue_g, 0)

    def drain_g(g, carry):
        sl = pl.ds(g * L, L)
        cellv = sv[sl] * N + dv[sl]
        pltpu.make_async_copy(kv_all.at[sl], claim_hbm.at[cellv], sem).wait()
        return carry

    lax.fori_loop(0, NGT, drain_g, 0)


def _sc_agg_body(src_hbm, dst_hbm, claim_hbm, inadj_hbm, hf_hbm, asf_hbm,
                 adf_hbm, zn_hbm, numf_hbm, denf_hbm,
                 sv, dv, cbuf, got, adjv, geff, shead, dhead, rows0, asb0,
                 adb0, wbuf0, rows1, asb1, adb1, wbuf1, num_sh, den_sh, sem0,
                 sem1, sema0, sema1):
    c = lax.axis_index("c")
    tid = lax.axis_index("s")
    wid = tid * NC + c
    base = wid * EPT
    pltpu.sync_copy(src_hbm.at[pl.ds(base, EPT)], sv)
    pltpu.sync_copy(dst_hbm.at[pl.ds(base, EPT)], dv)

    def cells_g(g, carry):
        sl = pl.ds(g * L, L)
        cbuf[sl] = sv[sl] * N + dv[sl]
        return carry

    lax.fori_loop(0, NGT, cells_g, 0)

    for t in range(EPT // GCH):
        wsl = pl.ds(t * GCH, GCH)
        pltpu.async_copy(claim_hbm.at[cbuf.at[wsl]], got.at[wsl], sem0).wait()
        pltpu.async_copy(inadj_hbm.at[cbuf.at[wsl]], adjv.at[wsl], sem0).wait()

    def geff_g(g, carry):
        sl = pl.ds(g * L, L)
        kv = base + g * L + lax.iota(jnp.int32, L)
        canon = got[sl] == kv
        svv = sv[sl]
        geff[sl] = jnp.where(canon,
                             adjv[sl] + jnp.where(svv == dv[sl], 1.0, 0.0),
                             0.0)
        return carry

    lax.fori_loop(0, NGT, geff_g, 0)

    myrows = pl.ds(tid * RPT, RPT)

    def head_body(i, carry):
        def dh_g(g, carry2):
            sl = pl.ds(g * L, L)
            shead[sl] = sv[sl] + (i + NH_TC) * N
            dhead[sl] = dv[sl] + (i + NH_TC) * N
            return carry2

        lax.fori_loop(0, NGT, dh_g, 0)

        pltpu.sync_copy(zn_hbm.at[myrows], num_sh.at[myrows])
        pltpu.sync_copy(zn_hbm.at[0], den_sh.at[pl.ds(tid * RPT, RPT)])
        plsc.subcore_barrier()

        def _issue(off, rows, asb, adb, semx):
            wsl = pl.ds(off, GCH)
            pltpu.async_copy(hf_hbm.at[dhead.at[wsl]], rows, semx)
            pltpu.async_copy(adf_hbm.at[dhead.at[wsl]], adb, semx)
            pltpu.async_copy(asf_hbm.at[shead.at[wsl]], asb, semx)

        def _process(off, rows, asb, adb, wbuf, semx, semax):
            wsl = pl.ds(off, GCH)
            pltpu.make_async_copy(hf_hbm.at[dhead.at[wsl]], rows, semx).wait()
            pltpu.make_async_copy(adf_hbm.at[dhead.at[wsl]], adb, semx).wait()
            pltpu.make_async_copy(asf_hbm.at[shead.at[wsl]], asb, semx).wait()
            adds = []
            for g in range(GCH // L):
                sl = pl.ds(off + g * L, L)
                gsl = pl.ds(g * L, L)
                w = jnp.exp(geff[sl] * _leaky(asb[gsl] + adb[gsl])) - 1.0
                wbuf[gsl] = w
                for eo in range(L):
                    ei = g * L + eo
                    wb = jnp.broadcast_to(w[eo], (L,))
                    for j in range(D // L):
                        rows[ei, pl.ds(j * L, L)] = (
                            rows[ei, pl.ds(j * L, L)] * wb)
                svv = sv[sl]
                adds.append(pltpu.async_copy(rows.at[gsl], num_sh.at[svv],
                                             semax, add=True))
                adds.append(pltpu.async_copy(wbuf.at[gsl], den_sh.at[svv],
                                             semax, add=True))
            for a in adds:
                a.wait()

        def chunk_body(q, carry2):
            @pl.when(jnp.logical_and(q < NCH, q % 2 == 0))
            def _i0():
                _issue(q * GCH, rows0, asb0, adb0, sem0)

            @pl.when(jnp.logical_and(q < NCH, q % 2 == 1))
            def _i1():
                _issue(q * GCH, rows1, asb1, adb1, sem1)

            @pl.when(jnp.logical_and(q > 0, (q - 1) % 2 == 0))
            def _p0():
                _process((q - 1) * GCH, rows0, asb0, adb0, wbuf0, sem0, sema0)

            @pl.when(jnp.logical_and(q > 0, (q - 1) % 2 == 1))
            def _p1():
                _process((q - 1) * GCH, rows1, asb1, adb1, wbuf1, sem1, sema1)

            return carry2

        lax.fori_loop(0, NCH + 1, chunk_body, 0)
        plsc.subcore_barrier()
        obase = (c * NH_SC + i) * N + tid * RPT
        pltpu.sync_copy(num_sh.at[myrows], numf_hbm.at[pl.ds(obase, RPT)])
        pltpu.sync_copy(den_sh.at[pl.ds(tid * RPT, RPT)],
                        denf_hbm.at[pl.ds(obase, RPT)])
        plsc.subcore_barrier()
        return carry

    lax.fori_loop(0, NH_SC, head_body, 0)


def _sc_aggregate(srcv, dstv, in_adj_flat, h, as_, ad_):
    mesh = plsc.VectorSubcoreMesh(core_axis_name="c", subcore_axis_name="s")
    claim_fn = pl.kernel(
        _sc_claim_body,
        out_type=jax.ShapeDtypeStruct((N * N,), jnp.int32),
        mesh=mesh,
        scratch_types=[
            pltpu.VMEM((EPT,), jnp.int32),
            pltpu.VMEM((EPT,), jnp.int32),
            pltpu.VMEM((EPT,), jnp.int32),
            pltpu.SemaphoreType.DMA,
        ],
    )
    claim = claim_fn(srcv, dstv)

    hf = h.reshape(NHEAD * N, D)
    asf = as_.reshape(NHEAD * N)
    adf = ad_.reshape(NHEAD * N)
    zn = jnp.zeros((N, D), jnp.float32)

    agg_fn = pl.kernel(
        _sc_agg_body,
        out_type=[
            jax.ShapeDtypeStruct((NC * NH_SC * N, D), jnp.float32),
            jax.ShapeDtypeStruct((NC * NH_SC * N,), jnp.float32),
        ],
        mesh=mesh,
        scratch_types=[
            pltpu.VMEM((EPT,), jnp.int32),
            pltpu.VMEM((EPT,), jnp.int32),
            pltpu.VMEM((EPT,), jnp.int32),
            pltpu.VMEM((EPT,), jnp.int32),
            pltpu.VMEM((EPT,), jnp.float32),
            pltpu.VMEM((EPT,), jnp.float32),
            pltpu.VMEM((EPT,), jnp.int32),
            pltpu.VMEM((EPT,), jnp.int32),
            pltpu.VMEM((GCH, D), jnp.float32),
            pltpu.VMEM((GCH,), jnp.float32),
            pltpu.VMEM((GCH,), jnp.float32),
            pltpu.VMEM((GCH,), jnp.float32),
            pltpu.VMEM((GCH, D), jnp.float32),
            pltpu.VMEM((GCH,), jnp.float32),
            pltpu.VMEM((GCH,), jnp.float32),
            pltpu.VMEM((GCH,), jnp.float32),
            pltpu.VMEM_SHARED((N, D), jnp.float32),
            pltpu.VMEM_SHARED((N,), jnp.float32),
            pltpu.SemaphoreType.DMA,
            pltpu.SemaphoreType.DMA,
            pltpu.SemaphoreType.DMA,
            pltpu.SemaphoreType.DMA,
        ],
    )
    numf, denf = agg_fn(srcv, dstv, claim, in_adj_flat, hf, asf, adf, zn)
    return (numf.reshape(NC, NH_SC, N, D), denf.reshape(NC, NH_SC, N))


def _proj_kernel(x_ref, w_ref, a1_ref, a2_ref, h_ref, as_ref, ad_ref, hsum_ref):
    r = pl.program_id(0)
    xb = x_ref[...]
    as_rows = []
    ad_rows = []
    hs = []
    for i in range(NHEAD):
        h = jnp.dot(xb, w_ref[i], preferred_element_type=jnp.float32)
        h_ref[i] = h
        as_rows.append(lax.dot_general(
            a1_ref[i][None, :], h, (((1,), (1,)), ((), ())),
            preferred_element_type=jnp.float32))
        ad_rows.append(lax.dot_general(
            a2_ref[i][None, :], h, (((1,), (1,)), ((), ())),
            preferred_element_type=jnp.float32))
        hs.append(jnp.sum(h, axis=0, keepdims=True))
    as_ref[...] = jnp.concatenate(as_rows, axis=0)
    ad_ref[...] = jnp.concatenate(ad_rows, axis=0)
    part = jnp.concatenate(hs, axis=0)

    @pl.when(r == 0)
    def _init():
        hsum_ref[...] = part

    @pl.when(r != 0)
    def _acc():
        hsum_ref[...] += part


def _heads_kernel(in_adj_ref, h_ref, as_ref, ad_ref, hsum_ref, b_ref,
                  h1a_ref, acc_ref, den_ref):
    r = pl.program_id(0)
    c = pl.program_id(1)
    rows = r * RB + jax.lax.broadcasted_iota(jnp.int32, (RB, CB), 0)
    cols = c * CB + jax.lax.broadcasted_iota(jnp.int32, (RB, CB), 1)
    adj = in_adj_ref[...] + jnp.where(rows == cols, 1.0, 0.0)

    dens = []
    for i in range(NH_TC):
        e = _leaky(as_ref[i][:, None] + ad_ref[i][None, :])
        w = jnp.exp(adj * e) - 1.0
        dens.append(jnp.sum(w, axis=1, keepdims=True))
        contrib = jnp.dot(w, h_ref[i], preferred_element_type=jnp.float32)

        @pl.when(c == 0)
        def _init(i=i, contrib=contrib):
            acc_ref[i] = contrib

        @pl.when(c != 0)
        def _acc(i=i, contrib=contrib):
            acc_ref[i] += contrib

    den_part = jnp.concatenate(dens, axis=1)

    @pl.when(c == 0)
    def _dinit():
        den_ref[...] = den_part

    @pl.when(c != 0)
    def _dacc():
        den_ref[...] += den_part

    @pl.when(c == NCB - 1)
    def _finalize():
        outs = []
        for i in range(NH_TC):
            numer = hsum_ref[i][None, :] + acc_ref[i]
            den = float(N) + den_ref[:, i][:, None]
            o = numer / den + b_ref[i][None, :]
            outs.append(jnp.where(o > 0, o, jnp.exp(o) - 1.0))
        h1a_ref[...] = jnp.concatenate(outs, axis=1)


def _hfin_kernel(num_ref, den_ref, hsum_ref, b_ref, h1_ref):
    outs = []
    for i in range(NH_SC):
        numer = hsum_ref[NH_TC + i][None, :] + num_ref[0, i] + num_ref[1, i]
        den = float(N) + den_ref[0, i, :] + den_ref[1, i, :]
        o = numer / den[:, None] + b_ref[NH_TC + i][None, :]
        outs.append(jnp.where(o > 0, o, jnp.exp(o) - 1.0))
    h1_ref[...] = jnp.concatenate(outs, axis=1)


def _out_proj_kernel(h1a_ref, h1b_ref, wa_ref, wb_ref, h2_ref, hsum2_ref):
    h2 = (jnp.dot(h1a_ref[...], wa_ref[...], preferred_element_type=jnp.float32)
          + jnp.dot(h1b_ref[...], wb_ref[...],
                    preferred_element_type=jnp.float32))
    h2_ref[...] = h2
    hsum2_ref[0] = jnp.sum(h2, axis=0, keepdims=True)


def _final_kernel(in_adj_ref, h2r_ref, h2c_ref, hsum2_ref, a1_ref, a2_ref,
                  b_ref, adj_ref, out_ref, acc_ref, den_ref):
    r = pl.program_id(0)
    c = pl.program_id(1)
    rows = r * RB + jax.lax.broadcasted_iota(jnp.int32, (RB, CB), 0)
    cols = c * CB + jax.lax.broadcasted_iota(jnp.int32, (RB, CB), 1)
    adj = in_adj_ref[...] + jnp.where(rows == cols, 1.0, 0.0)
    adj_ref[...] = adj

    h2r = h2r_ref[...]
    h2c = h2c_ref[...]
    asr = jnp.dot(h2r, a1_ref[...], preferred_element_type=jnp.float32)
    adc = jnp.dot(h2c, a2_ref[...], preferred_element_type=jnp.float32)
    e = _leaky(asr + adc[:, 0][None, :])
    w = jnp.exp(adj * e) - 1.0
    den_part = jnp.sum(w, axis=1, keepdims=True)
    contrib = jnp.dot(w, h2c, preferred_element_type=jnp.float32)

    @pl.when(c == 0)
    def _init():
        acc_ref[...] = contrib
        den_ref[...] = jnp.broadcast_to(den_part, (RB, 128))

    @pl.when(c != 0)
    def _acc():
        acc_ref[...] += contrib
        den_ref[...] += jnp.broadcast_to(den_part, (RB, 128))

    @pl.when(c == NCB - 1)
    def _finalize():
        hsum2 = jnp.sum(hsum2_ref[...], axis=0)
        numer = hsum2 + acc_ref[...]
        den = float(N) + den_ref[:, 0][:, None]
        o = numer / den + b_ref[...]
        lane = jax.lax.broadcasted_iota(jnp.int32, (RB, 128), 1)
        valid = lane < NCLASS
        om = jnp.where(valid, o, -jnp.inf)
        mx = jnp.max(om, axis=1, keepdims=True)
        ex = jnp.where(valid, jnp.exp(om - mx), 0.0)
        lse = jnp.log(jnp.sum(ex, axis=1, keepdims=True)) + mx
        out_ref[...] = jnp.where(valid, o - lse, 0.0)


def kernel(x, in_adj, edge_index, W_heads, a_heads, b_heads, W_out, a_out, b_out):
    a1 = a_heads[:, :D, 0]
    a2 = a_heads[:, D:, 0]

    h, as_, ad_, hsum = pl.pallas_call(
        _proj_kernel,
        grid=(NRB,),
        in_specs=[
            pl.BlockSpec((RB, D), lambda r: (r, 0)),
            pl.BlockSpec((NHEAD, D, D), lambda r: (0, 0, 0)),
            pl.BlockSpec((NHEAD, D), lambda r: (0, 0)),
            pl.BlockSpec((NHEAD, D), lambda r: (0, 0)),
        ],
        out_specs=[
            pl.BlockSpec((NHEAD, RB, D), lambda r: (0, r, 0)),
            pl.BlockSpec((NHEAD, RB), lambda r: (0, r)),
            pl.BlockSpec((NHEAD, RB), lambda r: (0, r)),
            pl.BlockSpec((NHEAD, D), lambda r: (0, 0)),
        ],
        out_shape=[
            jax.ShapeDtypeStruct((NHEAD, N, D), jnp.float32),
            jax.ShapeDtypeStruct((NHEAD, N), jnp.float32),
            jax.ShapeDtypeStruct((NHEAD, N), jnp.float32),
            jax.ShapeDtypeStruct((NHEAD, D), jnp.float32),
        ],
        compiler_params=pltpu.CompilerParams(
            dimension_semantics=("arbitrary",)),
    )(x, W_heads, a1, a2)

    loop = jnp.arange(N, dtype=jnp.int32)
    srcv = jnp.concatenate([edge_index[0].astype(jnp.int32), loop])
    dstv = jnp.concatenate([edge_index[1].astype(jnp.int32), loop])
    num, den = _sc_aggregate(srcv, dstv, in_adj.reshape(-1), h, as_, ad_)

    h1a = pl.pallas_call(
        _heads_kernel,
        grid=(NRB, NCB),
        in_specs=[
            pl.BlockSpec((RB, CB), lambda r, c: (r, c)),
            pl.BlockSpec((NH_TC, CB, D), lambda r, c: (0, c, 0)),
            pl.BlockSpec((NHEAD, RB), lambda r, c: (0, r)),
            pl.BlockSpec((NHEAD, CB), lambda r, c: (0, c)),
            pl.BlockSpec((NHEAD, D), lambda r, c: (0, 0)),
            pl.BlockSpec((NHEAD, D), lambda r, c: (0, 0)),
        ],
        out_specs=pl.BlockSpec((RB, NH_TC * D), lambda r, c: (r, 0)),
        out_shape=jax.ShapeDtypeStruct((N, NH_TC * D), jnp.float32),
        scratch_shapes=[
            pltpu.VMEM((NH_TC, RB, D), jnp.float32),
            pltpu.VMEM((RB, NH_TC), jnp.float32),
        ],
        compiler_params=pltpu.CompilerParams(
            dimension_semantics=("parallel", "arbitrary")),
    )(in_adj, h, as_, ad_, hsum, b_heads)

    h1b = pl.pallas_call(
        _hfin_kernel,
        grid=(NRB,),
        in_specs=[
            pl.BlockSpec((NC, NH_SC, RB, D), lambda r: (0, 0, r, 0)),
            pl.BlockSpec((NC, NH_SC, RB), lambda r: (0, 0, r)),
            pl.BlockSpec((NHEAD, D), lambda r: (0, 0)),
            pl.BlockSpec((NHEAD, D), lambda r: (0, 0)),
        ],
        out_specs=pl.BlockSpec((RB, NH_SC * D), lambda r: (r, 0)),
        out_shape=jax.ShapeDtypeStruct((N, NH_SC * D), jnp.float32),
        compiler_params=pltpu.CompilerParams(
            dimension_semantics=("parallel",)),
    )(num, den, hsum, b_heads)

    wout_pad = jnp.zeros((NHEAD * D, 128), jnp.float32).at[:, :NCLASS].set(W_out)
    wa = wout_pad[:NH_TC * D]
    wb = wout_pad[NH_TC * D:]
    a1o = jnp.zeros((128, 1), jnp.float32).at[:NCLASS, 0].set(a_out[:NCLASS, 0])
    a2o = jnp.zeros((128, 1), jnp.float32).at[:NCLASS, 0].set(a_out[NCLASS:, 0])
    bo = jnp.zeros((1, 128), jnp.float32).at[0, :NCLASS].set(b_out)

    h2, hsum2 = pl.pallas_call(
        _out_proj_kernel,
        grid=(NRB,),
        in_specs=[
            pl.BlockSpec((RB, NH_TC * D), lambda r: (r, 0)),
            pl.BlockSpec((RB, NH_SC * D), lambda r: (r, 0)),
            pl.BlockSpec((NH_TC * D, 128), lambda r: (0, 0)),
            pl.BlockSpec((NH_SC * D, 128), lambda r: (0, 0)),
        ],
        out_specs=[
            pl.BlockSpec((RB, 128), lambda r: (r, 0)),
            pl.BlockSpec((1, 1, 128), lambda r: (r, 0, 0)),
        ],
        out_shape=[
            jax.ShapeDtypeStruct((N, 128), jnp.float32),
            jax.ShapeDtypeStruct((NRB, 1, 128), jnp.float32),
        ],
        compiler_params=pltpu.CompilerParams(
            dimension_semantics=("arbitrary",)),
    )(h1a, h1b, wa, wb)

    adj, out_pad = pl.pallas_call(
        _final_kernel,
        grid=(NRB, NCB),
        in_specs=[
            pl.BlockSpec((RB, CB), lambda r, c: (r, c)),
            pl.BlockSpec((RB, 128), lambda r, c: (r, 0)),
            pl.BlockSpec((CB, 128), lambda r, c: (c, 0)),
            pl.BlockSpec((NRB, 1, 128), lambda r, c: (0, 0, 0)),
            pl.BlockSpec((128, 1), lambda r, c: (0, 0)),
            pl.BlockSpec((128, 1), lambda r, c: (0, 0)),
            pl.BlockSpec((1, 128), lambda r, c: (0, 0)),
        ],
        out_specs=[
            pl.BlockSpec((RB, CB), lambda r, c: (r, c)),
            pl.BlockSpec((RB, 128), lambda r, c: (r, 0)),
        ],
        out_shape=[
            jax.ShapeDtypeStruct((N, N), jnp.float32),
            jax.ShapeDtypeStruct((N, 128), jnp.float32),
        ],
        scratch_shapes=[
            pltpu.VMEM((RB, 128), jnp.float32),
            pltpu.VMEM((RB, 128), jnp.float32),
        ],
        compiler_params=pltpu.CompilerParams(
            dimension_semantics=("parallel", "arbitrary")),
    )(in_adj, h2, h2, hsum2, a1o, a2o, bo)

    return out_pad[:, :NCLASS], adj, x


# speedup vs baseline: 1.0322x; 1.0317x over previous
import jax
import jax.numpy as jnp
from jax import lax
from jax.experimental import pallas as pl
from jax.experimental.pallas import tpu as pltpu
from jax.experimental.pallas import tpu_sc as plsc

N = 2048
D = 128
NHEAD = 8
NHID = 128
NCLASS = 10
ALPHA = 0.2

RB = 256
CB = 256
NRB = N // RB
NCB = N // CB

NC = 2
NS = 16
L = 16
NW = NC * NS
E = 32768
E2 = E + N
EPT = E2 // NW
NGT = EPT // L
GCH = 64
NCH = EPT // GCH
RPT = N // NS

NH_TC = 6
NH_SC = NHEAD - NH_TC


def _leaky(v):
    return jnp.maximum(v, ALPHA * v)


def _sc_claim_body(src_hbm, dst_hbm, claim_hbm, sv, dv, kv_all, sem):
    c = lax.axis_index("c")
    tid = lax.axis_index("s")
    wid = tid * NC + c
    base = wid * EPT
    pltpu.sync_copy(src_hbm.at[pl.ds(base, EPT)], sv)
    pltpu.sync_copy(dst_hbm.at[pl.ds(base, EPT)], dv)

    def prep_g(g, carry):
        sl = pl.ds(g * L, L)
        kv_all[sl] = base + g * L + lax.iota(jnp.int32, L)
        return carry

    lax.fori_loop(0, NGT, prep_g, 0)

    def issue_g(g, carry):
        sl = pl.ds(g * L, L)
        cellv = sv[sl] * N + dv[sl]
        pltpu.async_copy(kv_all.at[sl], claim_hbm.at[cellv], sem)
        return carry

    lax.fori_loop(0, NGT, issue_g, 0)

    def drain_g(g, carry):
        sl = pl.ds(g * L, L)
        cellv = sv[sl] * N + dv[sl]
        pltpu.make_async_copy(kv_all.at[sl], claim_hbm.at[cellv], sem).wait()
        return carry

    lax.fori_loop(0, NGT, drain_g, 0)


def _sc_agg_body(src_hbm, dst_hbm, claim_hbm, inadj_hbm, hf_hbm, asf_hbm,
                 adf_hbm, zn_hbm, numf_hbm, denf_hbm,
                 sv, dv, cbuf, got, adjv, geff, shead, dhead, rows0, asb0,
                 adb0, wbuf0, rows1, asb1, adb1, wbuf1, num_sh, den_sh, sem0,
                 sem1, sema0, sema1):
    c = lax.axis_index("c")
    tid = lax.axis_index("s")
    wid = tid * NC + c
    base = wid * EPT
    pltpu.sync_copy(src_hbm.at[pl.ds(base, EPT)], sv)
    pltpu.sync_copy(dst_hbm.at[pl.ds(base, EPT)], dv)

    def cells_g(g, carry):
        sl = pl.ds(g * L, L)
        cbuf[sl] = sv[sl] * N + dv[sl]
        return carry

    lax.fori_loop(0, NGT, cells_g, 0)

    for t in range(EPT // GCH):
        wsl = pl.ds(t * GCH, GCH)
        pltpu.async_copy(claim_hbm.at[cbuf.at[wsl]], got.at[wsl], sem0).wait()
        pltpu.async_copy(inadj_hbm.at[cbuf.at[wsl]], adjv.at[wsl], sem0).wait()

    def geff_g(g, carry):
        sl = pl.ds(g * L, L)
        kv = base + g * L + lax.iota(jnp.int32, L)
        canon = got[sl] == kv
        svv = sv[sl]
        geff[sl] = jnp.where(canon,
                             adjv[sl] + jnp.where(svv == dv[sl], 1.0, 0.0),
                             0.0)
        return carry

    lax.fori_loop(0, NGT, geff_g, 0)

    myrows = pl.ds(tid * RPT, RPT)

    def head_body(i, carry):
        def dh_g(g, carry2):
            sl = pl.ds(g * L, L)
            shead[sl] = sv[sl] + (i + NH_TC) * N
            dhead[sl] = dv[sl] + (i + NH_TC) * N
            return carry2

        lax.fori_loop(0, NGT, dh_g, 0)

        pltpu.sync_copy(zn_hbm.at[myrows], num_sh.at[myrows])
        pltpu.sync_copy(zn_hbm.at[0], den_sh.at[pl.ds(tid * RPT, RPT)])
        plsc.subcore_barrier()

        def _issue(off, rows, asb, adb, semx):
            wsl = pl.ds(off, GCH)
            pltpu.async_copy(hf_hbm.at[dhead.at[wsl]], rows, semx)
            pltpu.async_copy(adf_hbm.at[dhead.at[wsl]], adb, semx)
            pltpu.async_copy(asf_hbm.at[shead.at[wsl]], asb, semx)

        def _process(off, rows, asb, adb, wbuf, semx, semax):
            wsl = pl.ds(off, GCH)
            pltpu.make_async_copy(hf_hbm.at[dhead.at[wsl]], rows, semx).wait()
            pltpu.make_async_copy(adf_hbm.at[dhead.at[wsl]], adb, semx).wait()
            pltpu.make_async_copy(asf_hbm.at[shead.at[wsl]], asb, semx).wait()
            adds = []
            for g in range(GCH // L):
                sl = pl.ds(off + g * L, L)
                gsl = pl.ds(g * L, L)
                w = jnp.exp(geff[sl] * _leaky(asb[gsl] + adb[gsl])) - 1.0
                wbuf[gsl] = w
                for eo in range(L):
                    ei = g * L + eo
                    wb = jnp.broadcast_to(w[eo], (L,))
                    for j in range(D // L):
                        rows[ei, pl.ds(j * L, L)] = (
                            rows[ei, pl.ds(j * L, L)] * wb)
                svv = sv[sl]
                adds.append(pltpu.async_copy(rows.at[gsl], num_sh.at[svv],
                                             semax, add=True))
                adds.append(pltpu.async_copy(wbuf.at[gsl], den_sh.at[svv],
                                             semax, add=True))
            for a in adds:
                a.wait()

        def chunk_body(q, carry2):
            @pl.when(jnp.logical_and(q < NCH, q % 2 == 0))
            def _i0():
                _issue(q * GCH, rows0, asb0, adb0, sem0)

            @pl.when(jnp.logical_and(q < NCH, q % 2 == 1))
            def _i1():
                _issue(q * GCH, rows1, asb1, adb1, sem1)

            @pl.when(jnp.logical_and(q > 0, (q - 1) % 2 == 0))
            def _p0():
                _process((q - 1) * GCH, rows0, asb0, adb0, wbuf0, sem0, sema0)

            @pl.when(jnp.logical_and(q > 0, (q - 1) % 2 == 1))
            def _p1():
                _process((q - 1) * GCH, rows1, asb1, adb1, wbuf1, sem1, sema1)

            return carry2

        lax.fori_loop(0, NCH + 1, chunk_body, 0)
        plsc.subcore_barrier()
        obase = (c * NH_SC + i) * N + tid * RPT
        pltpu.sync_copy(num_sh.at[myrows], numf_hbm.at[pl.ds(obase, RPT)])
        pltpu.sync_copy(den_sh.at[pl.ds(tid * RPT, RPT)],
                        denf_hbm.at[pl.ds(obase, RPT)])
        plsc.subcore_barrier()
        return carry

    lax.fori_loop(0, NH_SC, head_body, 0)


def _sc_aggregate(srcv, dstv, in_adj_flat, h, as_, ad_):
    mesh = plsc.VectorSubcoreMesh(core_axis_name="c", subcore_axis_name="s")
    claim_fn = pl.kernel(
        _sc_claim_body,
        out_type=jax.ShapeDtypeStruct((N * N,), jnp.int32),
        mesh=mesh,
        scratch_types=[
            pltpu.VMEM((EPT,), jnp.int32),
            pltpu.VMEM((EPT,), jnp.int32),
            pltpu.VMEM((EPT,), jnp.int32),
            pltpu.SemaphoreType.DMA,
        ],
    )
    claim = claim_fn(srcv, dstv)

    hf = h.reshape(NHEAD * N, D)
    asf = as_.reshape(NHEAD * N)
    adf = ad_.reshape(NHEAD * N)
    zn = jnp.zeros((N, D), jnp.float32)

    agg_fn = pl.kernel(
        _sc_agg_body,
        out_type=[
            jax.ShapeDtypeStruct((NC * NH_SC * N, D), jnp.float32),
            jax.ShapeDtypeStruct((NC * NH_SC * N,), jnp.float32),
        ],
        mesh=mesh,
        scratch_types=[
            pltpu.VMEM((EPT,), jnp.int32),
            pltpu.VMEM((EPT,), jnp.int32),
            pltpu.VMEM((EPT,), jnp.int32),
            pltpu.VMEM((EPT,), jnp.int32),
            pltpu.VMEM((EPT,), jnp.float32),
            pltpu.VMEM((EPT,), jnp.float32),
            pltpu.VMEM((EPT,), jnp.int32),
            pltpu.VMEM((EPT,), jnp.int32),
            pltpu.VMEM((GCH, D), jnp.float32),
            pltpu.VMEM((GCH,), jnp.float32),
            pltpu.VMEM((GCH,), jnp.float32),
            pltpu.VMEM((GCH,), jnp.float32),
            pltpu.VMEM((GCH, D), jnp.float32),
            pltpu.VMEM((GCH,), jnp.float32),
            pltpu.VMEM((GCH,), jnp.float32),
            pltpu.VMEM((GCH,), jnp.float32),
            pltpu.VMEM_SHARED((N, D), jnp.float32),
            pltpu.VMEM_SHARED((N,), jnp.float32),
            pltpu.SemaphoreType.DMA,
            pltpu.SemaphoreType.DMA,
            pltpu.SemaphoreType.DMA,
            pltpu.SemaphoreType.DMA,
        ],
    )
    numf, denf = agg_fn(srcv, dstv, claim, in_adj_flat, hf, asf, adf, zn)
    return (numf.reshape(NC, NH_SC, N, D), denf.reshape(NC, NH_SC, N))


def _proj_kernel(x_ref, w_ref, a1_ref, a2_ref, h_ref, as_ref, ad_ref, hsum_ref):
    r = pl.program_id(0)
    xb = x_ref[...]
    as_rows = []
    ad_rows = []
    hs = []
    for i in range(NHEAD):
        h = jnp.dot(xb, w_ref[i], preferred_element_type=jnp.float32)
        h_ref[i] = h
        as_rows.append(lax.dot_general(
            a1_ref[i][None, :], h, (((1,), (1,)), ((), ())),
            preferred_element_type=jnp.float32))
        ad_rows.append(lax.dot_general(
            a2_ref[i][None, :], h, (((1,), (1,)), ((), ())),
            preferred_element_type=jnp.float32))
        hs.append(jnp.sum(h, axis=0, keepdims=True))
    as_ref[...] = jnp.concatenate(as_rows, axis=0)
    ad_ref[...] = jnp.concatenate(ad_rows, axis=0)
    part = jnp.concatenate(hs, axis=0)

    @pl.when(r == 0)
    def _init():
        hsum_ref[...] = part

    @pl.when(r != 0)
    def _acc():
        hsum_ref[...] += part


def _heads_kernel(in_adj_ref, h_ref, as_ref, ad_ref, hsum_ref, b_ref,
                  h1a_ref, acc_ref, den_ref):
    r = pl.program_id(0)
    c = pl.program_id(1)
    rows = r * RB + jax.lax.broadcasted_iota(jnp.int32, (RB, CB), 0)
    cols = c * CB + jax.lax.broadcasted_iota(jnp.int32, (RB, CB), 1)
    adj = in_adj_ref[...] + jnp.where(rows == cols, 1.0, 0.0)

    dens = []
    for i in range(NH_TC):
        e = _leaky(as_ref[i][:, None] + ad_ref[i][None, :])
        w = jnp.exp(adj * e) - 1.0
        dens.append(jnp.sum(w, axis=1, keepdims=True))
        contrib = jnp.dot(w, h_ref[i, pl.ds(c * CB, CB), :],
                          preferred_element_type=jnp.float32)

        @pl.when(c == 0)
        def _init(i=i, contrib=contrib):
            acc_ref[i] = contrib

        @pl.when(c != 0)
        def _acc(i=i, contrib=contrib):
            acc_ref[i] += contrib

    den_part = jnp.concatenate(dens, axis=1)

    @pl.when(c == 0)
    def _dinit():
        den_ref[...] = den_part

    @pl.when(c != 0)
    def _dacc():
        den_ref[...] += den_part

    @pl.when(c == NCB - 1)
    def _finalize():
        outs = []
        for i in range(NH_TC):
            numer = hsum_ref[i][None, :] + acc_ref[i]
            den = float(N) + den_ref[:, i][:, None]
            o = numer / den + b_ref[i][None, :]
            outs.append(jnp.where(o > 0, o, jnp.exp(o) - 1.0))
        h1a_ref[...] = jnp.concatenate(outs, axis=1)


def _out_proj_kernel(h1a_ref, num_ref, den_ref, hsum_ref, b_ref, wa_ref,
                     wb_ref, h2_ref, hsum2_ref):
    outs = []
    for i in range(NH_SC):
        numer = hsum_ref[NH_TC + i][None, :] + num_ref[0, i] + num_ref[1, i]
        den = float(N) + den_ref[0, i, :] + den_ref[1, i, :]
        o = numer / den[:, None] + b_ref[NH_TC + i][None, :]
        outs.append(jnp.where(o > 0, o, jnp.exp(o) - 1.0))
    h1b = jnp.concatenate(outs, axis=1)
    h2 = (jnp.dot(h1a_ref[...], wa_ref[...], preferred_element_type=jnp.float32)
          + jnp.dot(h1b, wb_ref[...], preferred_element_type=jnp.float32))
    h2_ref[...] = h2
    hsum2_ref[0] = jnp.sum(h2, axis=0, keepdims=True)


def _final_kernel(in_adj_ref, h2r_ref, h2c_ref, hsum2_ref, a1_ref, a2_ref,
                  b_ref, adj_ref, out_ref, acc_ref, den_ref):
    r = pl.program_id(0)
    c = pl.program_id(1)
    rows = r * RB + jax.lax.broadcasted_iota(jnp.int32, (RB, CB), 0)
    cols = c * CB + jax.lax.broadcasted_iota(jnp.int32, (RB, CB), 1)
    adj = in_adj_ref[...] + jnp.where(rows == cols, 1.0, 0.0)
    adj_ref[...] = adj

    h2r = h2r_ref[...]
    h2c = h2c_ref[...]
    asr = jnp.dot(h2r, a1_ref[...], preferred_element_type=jnp.float32)
    adc = jnp.dot(h2c, a2_ref[...], preferred_element_type=jnp.float32)
    e = _leaky(asr + adc[:, 0][None, :])
    w = jnp.exp(adj * e) - 1.0
    den_part = jnp.sum(w, axis=1, keepdims=True)
    contrib = jnp.dot(w, h2c, preferred_element_type=jnp.float32)

    @pl.when(c == 0)
    def _init():
        acc_ref[...] = contrib
        den_ref[...] = jnp.broadcast_to(den_part, (RB, 128))

    @pl.when(c != 0)
    def _acc():
        acc_ref[...] += contrib
        den_ref[...] += jnp.broadcast_to(den_part, (RB, 128))

    @pl.when(c == NCB - 1)
    def _finalize():
        hsum2 = jnp.sum(hsum2_ref[...], axis=0)
        numer = hsum2 + acc_ref[...]
        den = float(N) + den_ref[:, 0][:, None]
        o = numer / den + b_ref[...]
        lane = jax.lax.broadcasted_iota(jnp.int32, (RB, 128), 1)
        valid = lane < NCLASS
        om = jnp.where(valid, o, -jnp.inf)
        mx = jnp.max(om, axis=1, keepdims=True)
        ex = jnp.where(valid, jnp.exp(om - mx), 0.0)
        lse = jnp.log(jnp.sum(ex, axis=1, keepdims=True)) + mx
        out_ref[...] = jnp.where(valid, o - lse, 0.0)


def kernel(x, in_adj, edge_index, W_heads, a_heads, b_heads, W_out, a_out, b_out):
    a1 = a_heads[:, :D, 0]
    a2 = a_heads[:, D:, 0]

    h, as_, ad_, hsum = pl.pallas_call(
        _proj_kernel,
        grid=(NRB,),
        in_specs=[
            pl.BlockSpec((RB, D), lambda r: (r, 0)),
            pl.BlockSpec((NHEAD, D, D), lambda r: (0, 0, 0)),
            pl.BlockSpec((NHEAD, D), lambda r: (0, 0)),
            pl.BlockSpec((NHEAD, D), lambda r: (0, 0)),
        ],
        out_specs=[
            pl.BlockSpec((NHEAD, RB, D), lambda r: (0, r, 0)),
            pl.BlockSpec((NHEAD, RB), lambda r: (0, r)),
            pl.BlockSpec((NHEAD, RB), lambda r: (0, r)),
            pl.BlockSpec((NHEAD, D), lambda r: (0, 0)),
        ],
        out_shape=[
            jax.ShapeDtypeStruct((NHEAD, N, D), jnp.float32),
            jax.ShapeDtypeStruct((NHEAD, N), jnp.float32),
            jax.ShapeDtypeStruct((NHEAD, N), jnp.float32),
            jax.ShapeDtypeStruct((NHEAD, D), jnp.float32),
        ],
        compiler_params=pltpu.CompilerParams(
            dimension_semantics=("arbitrary",)),
    )(x, W_heads, a1, a2)

    loop = jnp.arange(N, dtype=jnp.int32)
    srcv = jnp.concatenate([edge_index[0].astype(jnp.int32), loop])
    dstv = jnp.concatenate([edge_index[1].astype(jnp.int32), loop])
    num, den = _sc_aggregate(srcv, dstv, in_adj.reshape(-1), h, as_, ad_)

    h1a = pl.pallas_call(
        _heads_kernel,
        grid=(NRB, NCB),
        in_specs=[
            pl.BlockSpec((RB, CB), lambda r, c: (r, c)),
            pl.BlockSpec((NH_TC, N, D), lambda r, c: (0, 0, 0)),
            pl.BlockSpec((NHEAD, RB), lambda r, c: (0, r)),
            pl.BlockSpec((NHEAD, CB), lambda r, c: (0, c)),
            pl.BlockSpec((NHEAD, D), lambda r, c: (0, 0)),
            pl.BlockSpec((NHEAD, D), lambda r, c: (0, 0)),
        ],
        out_specs=pl.BlockSpec((RB, NH_TC * D), lambda r, c: (r, 0)),
        out_shape=jax.ShapeDtypeStruct((N, NH_TC * D), jnp.float32),
        scratch_shapes=[
            pltpu.VMEM((NH_TC, RB, D), jnp.float32),
            pltpu.VMEM((RB, NH_TC), jnp.float32),
        ],
        compiler_params=pltpu.CompilerParams(
            dimension_semantics=("parallel", "arbitrary")),
    )(in_adj, h, as_, ad_, hsum, b_heads)

    wout_pad = jnp.zeros((NHEAD * D, 128), jnp.float32).at[:, :NCLASS].set(W_out)
    wa = wout_pad[:NH_TC * D]
    wb = wout_pad[NH_TC * D:]
    a1o = jnp.zeros((128, 1), jnp.float32).at[:NCLASS, 0].set(a_out[:NCLASS, 0])
    a2o = jnp.zeros((128, 1), jnp.float32).at[:NCLASS, 0].set(a_out[NCLASS:, 0])
    bo = jnp.zeros((1, 128), jnp.float32).at[0, :NCLASS].set(b_out)

    h2, hsum2 = pl.pallas_call(
        _out_proj_kernel,
        grid=(NRB,),
        in_specs=[
            pl.BlockSpec((RB, NH_TC * D), lambda r: (r, 0)),
            pl.BlockSpec((NC, NH_SC, RB, D), lambda r: (0, 0, r, 0)),
            pl.BlockSpec((NC, NH_SC, RB), lambda r: (0, 0, r)),
            pl.BlockSpec((NHEAD, D), lambda r: (0, 0)),
            pl.BlockSpec((NHEAD, D), lambda r: (0, 0)),
            pl.BlockSpec((NH_TC * D, 128), lambda r: (0, 0)),
            pl.BlockSpec((NH_SC * D, 128), lambda r: (0, 0)),
        ],
        out_specs=[
            pl.BlockSpec((RB, 128), lambda r: (r, 0)),
            pl.BlockSpec((1, 1, 128), lambda r: (r, 0, 0)),
        ],
        out_shape=[
            jax.ShapeDtypeStruct((N, 128), jnp.float32),
            jax.ShapeDtypeStruct((NRB, 1, 128), jnp.float32),
        ],
        compiler_params=pltpu.CompilerParams(
            dimension_semantics=("arbitrary",)),
    )(h1a, num, den, hsum, b_heads, wa, wb)

    adj, out_pad = pl.pallas_call(
        _final_kernel,
        grid=(NRB, NCB),
        in_specs=[
            pl.BlockSpec((RB, CB), lambda r, c: (r, c)),
            pl.BlockSpec((RB, 128), lambda r, c: (r, 0)),
            pl.BlockSpec((CB, 128), lambda r, c: (c, 0)),
            pl.BlockSpec((NRB, 1, 128), lambda r, c: (0, 0, 0)),
            pl.BlockSpec((128, 1), lambda r, c: (0, 0)),
            pl.BlockSpec((128, 1), lambda r, c: (0, 0)),
            pl.BlockSpec((1, 128), lambda r, c: (0, 0)),
        ],
        out_specs=[
            pl.BlockSpec((RB, CB), lambda r, c: (r, c)),
            pl.BlockSpec((RB, 128), lambda r, c: (r, 0)),
        ],
        out_shape=[
            jax.ShapeDtypeStruct((N, N), jnp.float32),
            jax.ShapeDtypeStruct((N, 128), jnp.float32),
        ],
        scratch_shapes=[
            pltpu.VMEM((RB, 128), jnp.float32),
            pltpu.VMEM((RB, 128), jnp.float32),
        ],
        compiler_params=pltpu.CompilerParams(
            dimension_semantics=("parallel", "arbitrary")),
    )(in_adj, h2, h2, hsum2, a1o, a2o, bo)

    return out_pad[:, :NCLASS], adj, x


# speedup vs baseline: 1.0463x; 1.0136x over previous
import jax
import jax.numpy as jnp
from jax import lax
from jax.experimental import pallas as pl
from jax.experimental.pallas import tpu as pltpu
from jax.experimental.pallas import tpu_sc as plsc

N = 2048
D = 128
NHEAD = 8
NHID = 128
NCLASS = 10
ALPHA = 0.2

RB = 256
CB = 256
NRB = N // RB
NCB = N // CB

NC = 2
NS = 16
L = 16
NW = NC * NS
E = 32768
E2 = E + N
EPT = E2 // NW
NGT = EPT // L
GCH = 64
NCH = EPT // GCH
RPT = N // NS

NH_TC = 6
NH_SC = NHEAD - NH_TC


def _leaky(v):
    return jnp.maximum(v, ALPHA * v)


def _sc_claim_body(src_hbm, dst_hbm, claim_hbm, sv, dv, kv_all, sem):
    c = lax.axis_index("c")
    tid = lax.axis_index("s")
    wid = tid * NC + c
    base = wid * EPT
    pltpu.sync_copy(src_hbm.at[pl.ds(base, EPT)], sv)
    pltpu.sync_copy(dst_hbm.at[pl.ds(base, EPT)], dv)

    def prep_g(g, carry):
        sl = pl.ds(g * L, L)
        kv_all[sl] = base + g * L + lax.iota(jnp.int32, L)
        return carry

    lax.fori_loop(0, NGT, prep_g, 0)

    def issue_g(g, carry):
        sl = pl.ds(g * L, L)
        cellv = sv[sl] * N + dv[sl]
        pltpu.async_copy(kv_all.at[sl], claim_hbm.at[cellv], sem)
        return carry

    lax.fori_loop(0, NGT, issue_g, 0)

    def drain_g(g, carry):
        sl = pl.ds(g * L, L)
        cellv = sv[sl] * N + dv[sl]
        pltpu.make_async_copy(kv_all.at[sl], claim_hbm.at[cellv], sem).wait()
        return carry

    lax.fori_loop(0, NGT, drain_g, 0)


def _sc_agg_body(src_hbm, dst_hbm, claim_hbm, inadj_hbm, hf_hbm, asf_hbm,
                 adf_hbm, zn_hbm, numf_hbm, denf_hbm,
                 sv, dv, cbuf, got, adjv, geff, shead, dhead, rows0, asb0,
                 adb0, wbuf0, rows1, asb1, adb1, wbuf1, num_sh, den_sh, sem0,
                 sem1, sema0, sema1):
    c = lax.axis_index("c")
    tid = lax.axis_index("s")
    wid = tid * NC + c
    base = wid * EPT
    pltpu.sync_copy(src_hbm.at[pl.ds(base, EPT)], sv)
    pltpu.sync_copy(dst_hbm.at[pl.ds(base, EPT)], dv)

    def cells_g(g, carry):
        sl = pl.ds(g * L, L)
        cbuf[sl] = sv[sl] * N + dv[sl]
        return carry

    lax.fori_loop(0, NGT, cells_g, 0)

    for t in range(EPT // GCH):
        wsl = pl.ds(t * GCH, GCH)
        pltpu.async_copy(claim_hbm.at[cbuf.at[wsl]], got.at[wsl], sem0).wait()
        pltpu.async_copy(inadj_hbm.at[cbuf.at[wsl]], adjv.at[wsl], sem0).wait()

    def geff_g(g, carry):
        sl = pl.ds(g * L, L)
        kv = base + g * L + lax.iota(jnp.int32, L)
        canon = got[sl] == kv
        svv = sv[sl]
        geff[sl] = jnp.where(canon,
                             adjv[sl] + jnp.where(svv == dv[sl], 1.0, 0.0),
                             0.0)
        return carry

    lax.fori_loop(0, NGT, geff_g, 0)

    myrows = pl.ds(tid * RPT, RPT)

    def head_body(i, carry):
        def dh_g(g, carry2):
            sl = pl.ds(g * L, L)
            shead[sl] = sv[sl] + (i + NH_TC) * N
            dhead[sl] = dv[sl] + (i + NH_TC) * N
            return carry2

        lax.fori_loop(0, NGT, dh_g, 0)

        pltpu.sync_copy(zn_hbm.at[myrows], num_sh.at[myrows])
        pltpu.sync_copy(zn_hbm.at[0], den_sh.at[pl.ds(tid * RPT, RPT)])
        plsc.subcore_barrier()

        def _issue(off, rows, asb, adb, semx):
            wsl = pl.ds(off, GCH)
            pltpu.async_copy(hf_hbm.at[dhead.at[wsl]], rows, semx)
            pltpu.async_copy(adf_hbm.at[dhead.at[wsl]], adb, semx)
            pltpu.async_copy(asf_hbm.at[shead.at[wsl]], asb, semx)

        def _process(off, rows, asb, adb, wbuf, semx, semax):
            wsl = pl.ds(off, GCH)
            pltpu.make_async_copy(hf_hbm.at[dhead.at[wsl]], rows, semx).wait()
            pltpu.make_async_copy(adf_hbm.at[dhead.at[wsl]], adb, semx).wait()
            pltpu.make_async_copy(asf_hbm.at[shead.at[wsl]], asb, semx).wait()
            adds = []
            for g in range(GCH // L):
                sl = pl.ds(off + g * L, L)
                gsl = pl.ds(g * L, L)
                w = jnp.exp(geff[sl] * _leaky(asb[gsl] + adb[gsl])) - 1.0
                wbuf[gsl] = w
                for eo in range(L):
                    ei = g * L + eo
                    wb = jnp.broadcast_to(w[eo], (L,))
                    for j in range(D // L):
                        rows[ei, pl.ds(j * L, L)] = (
                            rows[ei, pl.ds(j * L, L)] * wb)
                svv = sv[sl]
                adds.append(pltpu.async_copy(rows.at[gsl], num_sh.at[svv],
                                             semax, add=True))
                adds.append(pltpu.async_copy(wbuf.at[gsl], den_sh.at[svv],
                                             semax, add=True))
            for a in adds:
                a.wait()

        def chunk_body(q, carry2):
            @pl.when(jnp.logical_and(q < NCH, q % 2 == 0))
            def _i0():
                _issue(q * GCH, rows0, asb0, adb0, sem0)

            @pl.when(jnp.logical_and(q < NCH, q % 2 == 1))
            def _i1():
                _issue(q * GCH, rows1, asb1, adb1, sem1)

            @pl.when(jnp.logical_and(q > 0, (q - 1) % 2 == 0))
            def _p0():
                _process((q - 1) * GCH, rows0, asb0, adb0, wbuf0, sem0, sema0)

            @pl.when(jnp.logical_and(q > 0, (q - 1) % 2 == 1))
            def _p1():
                _process((q - 1) * GCH, rows1, asb1, adb1, wbuf1, sem1, sema1)

            return carry2

        lax.fori_loop(0, NCH + 1, chunk_body, 0)
        plsc.subcore_barrier()
        obase = (c * NH_SC + i) * N + tid * RPT
        pltpu.sync_copy(num_sh.at[myrows], numf_hbm.at[pl.ds(obase, RPT)])
        pltpu.sync_copy(den_sh.at[pl.ds(tid * RPT, RPT)],
                        denf_hbm.at[pl.ds(obase, RPT)])
        plsc.subcore_barrier()
        return carry

    lax.fori_loop(0, NH_SC, head_body, 0)


def _sc_aggregate(srcv, dstv, in_adj_flat, h, as_, ad_):
    mesh = plsc.VectorSubcoreMesh(core_axis_name="c", subcore_axis_name="s")
    claim_fn = pl.kernel(
        _sc_claim_body,
        out_type=jax.ShapeDtypeStruct((N * N,), jnp.int32),
        mesh=mesh,
        scratch_types=[
            pltpu.VMEM((EPT,), jnp.int32),
            pltpu.VMEM((EPT,), jnp.int32),
            pltpu.VMEM((EPT,), jnp.int32),
            pltpu.SemaphoreType.DMA,
        ],
    )
    claim = claim_fn(srcv, dstv)

    hf = h.reshape(NHEAD * N, D)
    asf = as_.reshape(NHEAD * N)
    adf = ad_.reshape(NHEAD * N)
    zn = jnp.zeros((N, D), jnp.float32)

    agg_fn = pl.kernel(
        _sc_agg_body,
        out_type=[
            jax.ShapeDtypeStruct((NC * NH_SC * N, D), jnp.float32),
            jax.ShapeDtypeStruct((NC * NH_SC * N,), jnp.float32),
        ],
        mesh=mesh,
        scratch_types=[
            pltpu.VMEM((EPT,), jnp.int32),
            pltpu.VMEM((EPT,), jnp.int32),
            pltpu.VMEM((EPT,), jnp.int32),
            pltpu.VMEM((EPT,), jnp.int32),
            pltpu.VMEM((EPT,), jnp.float32),
            pltpu.VMEM((EPT,), jnp.float32),
            pltpu.VMEM((EPT,), jnp.int32),
            pltpu.VMEM((EPT,), jnp.int32),
            pltpu.VMEM((GCH, D), jnp.float32),
            pltpu.VMEM((GCH,), jnp.float32),
            pltpu.VMEM((GCH,), jnp.float32),
            pltpu.VMEM((GCH,), jnp.float32),
            pltpu.VMEM((GCH, D), jnp.float32),
            pltpu.VMEM((GCH,), jnp.float32),
            pltpu.VMEM((GCH,), jnp.float32),
            pltpu.VMEM((GCH,), jnp.float32),
            pltpu.VMEM_SHARED((N, D), jnp.float32),
            pltpu.VMEM_SHARED((N,), jnp.float32),
            pltpu.SemaphoreType.DMA,
            pltpu.SemaphoreType.DMA,
            pltpu.SemaphoreType.DMA,
            pltpu.SemaphoreType.DMA,
        ],
    )
    numf, denf = agg_fn(srcv, dstv, claim, in_adj_flat, hf, asf, adf, zn)
    return (numf.reshape(NC, NH_SC, N, D), denf.reshape(NC, NH_SC, N))


def _proj_kernel(x_ref, w_ref, a1_ref, a2_ref, h_ref, as_ref, ad_ref, hsum_ref):
    r = pl.program_id(0)
    xb = x_ref[...]
    as_rows = []
    ad_rows = []
    hs = []
    for i in range(NHEAD):
        h = jnp.dot(xb, w_ref[i], preferred_element_type=jnp.float32)
        h_ref[i] = h
        as_rows.append(lax.dot_general(
            a1_ref[i][None, :], h, (((1,), (1,)), ((), ())),
            preferred_element_type=jnp.float32))
        ad_rows.append(lax.dot_general(
            a2_ref[i][None, :], h, (((1,), (1,)), ((), ())),
            preferred_element_type=jnp.float32))
        hs.append(jnp.sum(h, axis=0, keepdims=True))
    as_ref[...] = jnp.concatenate(as_rows, axis=0)
    ad_ref[...] = jnp.concatenate(ad_rows, axis=0)
    part = jnp.concatenate(hs, axis=0)

    @pl.when(r == 0)
    def _init():
        hsum_ref[...] = part

    @pl.when(r != 0)
    def _acc():
        hsum_ref[...] += part


def _heads_kernel(in_adj_ref, h_ref, as_ref, ad_ref, hsum_ref, b_ref,
                  h1a_ref, acc_ref, den_ref):
    r = pl.program_id(0)
    c = pl.program_id(1)
    rows = r * RB + jax.lax.broadcasted_iota(jnp.int32, (RB, CB), 0)
    cols = c * CB + jax.lax.broadcasted_iota(jnp.int32, (RB, CB), 1)
    adj = in_adj_ref[...] + jnp.where(rows == cols, 1.0, 0.0)

    dens = []
    for i in range(NH_TC):
        e = _leaky(as_ref[i][:, None] + ad_ref[i][None, :])
        w = jnp.exp(adj * e) - 1.0
        dens.append(jnp.sum(w, axis=1, keepdims=True))
        contrib = jnp.dot(w, h_ref[i, pl.ds(c * CB, CB), :],
                          preferred_element_type=jnp.float32)

        @pl.when(c == 0)
        def _init(i=i, contrib=contrib):
            acc_ref[i] = contrib

        @pl.when(c != 0)
        def _acc(i=i, contrib=contrib):
            acc_ref[i] += contrib

    den_part = jnp.concatenate(dens, axis=1)

    @pl.when(c == 0)
    def _dinit():
        den_ref[...] = den_part

    @pl.when(c != 0)
    def _dacc():
        den_ref[...] += den_part

    @pl.when(c == NCB - 1)
    def _finalize():
        outs = []
        for i in range(NH_TC):
            numer = hsum_ref[i][None, :] + acc_ref[i]
            den = float(N) + den_ref[:, i][:, None]
            o = numer / den + b_ref[i][None, :]
            outs.append(jnp.where(o > 0, o, jnp.exp(o) - 1.0))
        h1a_ref[...] = jnp.concatenate(outs, axis=1)


def _out_proj_kernel(h1a_ref, num_ref, den_ref, hsum_ref, b_ref, wa_ref,
                     wb_ref, h2_ref, hsum2_ref):
    outs = []
    for i in range(NH_SC):
        numer = hsum_ref[NH_TC + i][None, :] + num_ref[0, i] + num_ref[1, i]
        den = float(N) + den_ref[0, i, :] + den_ref[1, i, :]
        o = numer / den[:, None] + b_ref[NH_TC + i][None, :]
        outs.append(jnp.where(o > 0, o, jnp.exp(o) - 1.0))
    h1b = jnp.concatenate(outs, axis=1)
    h2 = (jnp.dot(h1a_ref[...], wa_ref[...], preferred_element_type=jnp.float32)
          + jnp.dot(h1b, wb_ref[...], preferred_element_type=jnp.float32))
    h2_ref[...] = h2
    hsum2_ref[0] = jnp.sum(h2, axis=0, keepdims=True)


def _final_kernel(in_adj_ref, h2_ref, hsum2_ref, a1_ref, a2_ref,
                  b_ref, adj_ref, out_ref, acc_ref, den_ref):
    r = pl.program_id(0)
    c = pl.program_id(1)
    rows = r * RB + jax.lax.broadcasted_iota(jnp.int32, (RB, CB), 0)
    cols = c * CB + jax.lax.broadcasted_iota(jnp.int32, (RB, CB), 1)
    adj = in_adj_ref[...] + jnp.where(rows == cols, 1.0, 0.0)
    adj_ref[...] = adj

    h2r = h2_ref[pl.ds(r * RB, RB), :]
    h2c = h2_ref[pl.ds(c * CB, CB), :]
    asr = jnp.dot(h2r, a1_ref[...], preferred_element_type=jnp.float32)
    adc = jnp.dot(h2c, a2_ref[...], preferred_element_type=jnp.float32)
    e = _leaky(asr + adc[:, 0][None, :])
    w = jnp.exp(adj * e) - 1.0
    den_part = jnp.sum(w, axis=1, keepdims=True)
    contrib = jnp.dot(w, h2c, preferred_element_type=jnp.float32)

    @pl.when(c == 0)
    def _init():
        acc_ref[...] = contrib
        den_ref[...] = jnp.broadcast_to(den_part, (RB, 128))

    @pl.when(c != 0)
    def _acc():
        acc_ref[...] += contrib
        den_ref[...] += jnp.broadcast_to(den_part, (RB, 128))

    @pl.when(c == NCB - 1)
    def _finalize():
        hsum2 = jnp.sum(hsum2_ref[...], axis=0)
        numer = hsum2 + acc_ref[...]
        den = float(N) + den_ref[:, 0][:, None]
        o = numer / den + b_ref[...]
        lane = jax.lax.broadcasted_iota(jnp.int32, (RB, 128), 1)
        valid = lane < NCLASS
        om = jnp.where(valid, o, -jnp.inf)
        mx = jnp.max(om, axis=1, keepdims=True)
        ex = jnp.where(valid, jnp.exp(om - mx), 0.0)
        lse = jnp.log(jnp.sum(ex, axis=1, keepdims=True)) + mx
        out_ref[...] = jnp.where(valid, o - lse, 0.0)


def kernel(x, in_adj, edge_index, W_heads, a_heads, b_heads, W_out, a_out, b_out):
    a1 = a_heads[:, :D, 0]
    a2 = a_heads[:, D:, 0]

    h, as_, ad_, hsum = pl.pallas_call(
        _proj_kernel,
        grid=(NRB,),
        in_specs=[
            pl.BlockSpec((RB, D), lambda r: (r, 0)),
            pl.BlockSpec((NHEAD, D, D), lambda r: (0, 0, 0)),
            pl.BlockSpec((NHEAD, D), lambda r: (0, 0)),
            pl.BlockSpec((NHEAD, D), lambda r: (0, 0)),
        ],
        out_specs=[
            pl.BlockSpec((NHEAD, RB, D), lambda r: (0, r, 0)),
            pl.BlockSpec((NHEAD, RB), lambda r: (0, r)),
            pl.BlockSpec((NHEAD, RB), lambda r: (0, r)),
            pl.BlockSpec((NHEAD, D), lambda r: (0, 0)),
        ],
        out_shape=[
            jax.ShapeDtypeStruct((NHEAD, N, D), jnp.float32),
            jax.ShapeDtypeStruct((NHEAD, N), jnp.float32),
            jax.ShapeDtypeStruct((NHEAD, N), jnp.float32),
            jax.ShapeDtypeStruct((NHEAD, D), jnp.float32),
        ],
        compiler_params=pltpu.CompilerParams(
            dimension_semantics=("arbitrary",)),
    )(x, W_heads, a1, a2)

    loop = jnp.arange(N, dtype=jnp.int32)
    srcv = jnp.concatenate([edge_index[0].astype(jnp.int32), loop])
    dstv = jnp.concatenate([edge_index[1].astype(jnp.int32), loop])
    num, den = _sc_aggregate(srcv, dstv, in_adj.reshape(-1), h, as_, ad_)

    h1a = pl.pallas_call(
        _heads_kernel,
        grid=(NRB, NCB),
        in_specs=[
            pl.BlockSpec((RB, CB), lambda r, c: (r, c)),
            pl.BlockSpec((NH_TC, N, D), lambda r, c: (0, 0, 0)),
            pl.BlockSpec((NHEAD, RB), lambda r, c: (0, r)),
            pl.BlockSpec((NHEAD, CB), lambda r, c: (0, c)),
            pl.BlockSpec((NHEAD, D), lambda r, c: (0, 0)),
            pl.BlockSpec((NHEAD, D), lambda r, c: (0, 0)),
        ],
        out_specs=pl.BlockSpec((RB, NH_TC * D), lambda r, c: (r, 0)),
        out_shape=jax.ShapeDtypeStruct((N, NH_TC * D), jnp.float32),
        scratch_shapes=[
            pltpu.VMEM((NH_TC, RB, D), jnp.float32),
            pltpu.VMEM((RB, NH_TC), jnp.float32),
        ],
        compiler_params=pltpu.CompilerParams(
            dimension_semantics=("parallel", "arbitrary")),
    )(in_adj, h, as_, ad_, hsum, b_heads)

    wout_pad = jnp.zeros((NHEAD * D, 128), jnp.float32).at[:, :NCLASS].set(W_out)
    wa = wout_pad[:NH_TC * D]
    wb = wout_pad[NH_TC * D:]
    a1o = jnp.zeros((128, 1), jnp.float32).at[:NCLASS, 0].set(a_out[:NCLASS, 0])
    a2o = jnp.zeros((128, 1), jnp.float32).at[:NCLASS, 0].set(a_out[NCLASS:, 0])
    bo = jnp.zeros((1, 128), jnp.float32).at[0, :NCLASS].set(b_out)

    h2, hsum2 = pl.pallas_call(
        _out_proj_kernel,
        grid=(NRB,),
        in_specs=[
            pl.BlockSpec((RB, NH_TC * D), lambda r: (r, 0)),
            pl.BlockSpec((NC, NH_SC, RB, D), lambda r: (0, 0, r, 0)),
            pl.BlockSpec((NC, NH_SC, RB), lambda r: (0, 0, r)),
            pl.BlockSpec((NHEAD, D), lambda r: (0, 0)),
            pl.BlockSpec((NHEAD, D), lambda r: (0, 0)),
            pl.BlockSpec((NH_TC * D, 128), lambda r: (0, 0)),
            pl.BlockSpec((NH_SC * D, 128), lambda r: (0, 0)),
        ],
        out_specs=[
            pl.BlockSpec((RB, 128), lambda r: (r, 0)),
            pl.BlockSpec((1, 1, 128), lambda r: (r, 0, 0)),
        ],
        out_shape=[
            jax.ShapeDtypeStruct((N, 128), jnp.float32),
            jax.ShapeDtypeStruct((NRB, 1, 128), jnp.float32),
        ],
        compiler_params=pltpu.CompilerParams(
            dimension_semantics=("arbitrary",)),
    )(h1a, num, den, hsum, b_heads, wa, wb)

    adj, out_pad = pl.pallas_call(
        _final_kernel,
        grid=(NRB, NCB),
        in_specs=[
            pl.BlockSpec((RB, CB), lambda r, c: (r, c)),
            pl.BlockSpec((N, 128), lambda r, c: (0, 0)),
            pl.BlockSpec((NRB, 1, 128), lambda r, c: (0, 0, 0)),
            pl.BlockSpec((128, 1), lambda r, c: (0, 0)),
            pl.BlockSpec((128, 1), lambda r, c: (0, 0)),
            pl.BlockSpec((1, 128), lambda r, c: (0, 0)),
        ],
        out_specs=[
            pl.BlockSpec((RB, CB), lambda r, c: (r, c)),
            pl.BlockSpec((RB, 128), lambda r, c: (r, 0)),
        ],
        out_shape=[
            jax.ShapeDtypeStruct((N, N), jnp.float32),
            jax.ShapeDtypeStruct((N, 128), jnp.float32),
        ],
        scratch_shapes=[
            pltpu.VMEM((RB, 128), jnp.float32),
            pltpu.VMEM((RB, 128), jnp.float32),
        ],
        compiler_params=pltpu.CompilerParams(
            dimension_semantics=("parallel", "arbitrary")),
    )(in_adj, h2, hsum2, a1o, a2o, bo)

    return out_pad[:, :NCLASS], adj, x
